# baseline (speedup 1.0000x reference)
import functools

import jax
import jax.numpy as jnp
from jax import lax
from jax.experimental import pallas as pl
from jax.experimental.pallas import tpu as pltpu
from jax.experimental.pallas import tpu_sc as plsc

B = 16384
D = 64
NC = 2
NS = 16
NW = NC * NS
BPW = B // NW
L = 16
CH = 128
NCH = BPW // CH
CHG = CH // L


def _sc_body(uidx_hbm, pidx_hbm, uemb_hbm, pemb_hbm, ubias_hbm, pbias_hbm,
             out_hbm, uidx_v, pidx_v, urows_v, prows_v, ub_v, pb_v, out_v,
             sem, sem2):
    c = lax.axis_index("c")
    s = lax.axis_index("s")
    wid = s * NC + c
    base = wid * BPW

    pltpu.sync_copy(uidx_hbm.at[pl.ds(base, BPW)], uidx_v)
    pltpu.sync_copy(pidx_hbm.at[pl.ds(base, BPW)], pidx_v)

    last_lane = lax.iota(jnp.int32, L) == (L - 1)
    lanes = lax.iota(jnp.int32, L)
    zeros = jnp.zeros((L,), jnp.int32)

    def chunk(cc, carry):
        cbase = cc * CH

        def issue(g, carry):
            iu = uidx_v[pl.ds(cbase + g * L, L)]
            ip = pidx_v[pl.ds(cbase + g * L, L)]
            for i in range(L):
                rl = g * L + i
                pltpu.async_copy(uemb_hbm.at[pl.ds(iu[i], 1)],
                                 urows_v.at[pl.ds(rl, 1)], sem)
                pltpu.async_copy(pemb_hbm.at[pl.ds(ip[i], 1)],
                                 prows_v.at[pl.ds(rl, 1)], sem)
                pltpu.async_copy(ubias_hbm.at[pl.ds(iu[i], 1)],
                                 ub_v.at[pl.ds(rl, 1)], sem2)
                pltpu.async_copy(pbias_hbm.at[pl.ds(ip[i], 1)],
                                 pb_v.at[pl.ds(rl, 1)], sem2)
            return carry

        lax.fori_loop(0, CHG, issue, 0)
        pltpu.make_async_copy(uemb_hbm.at[pl.ds(0, CH)], urows_v, sem).wait()
        pltpu.make_async_copy(pemb_hbm.at[pl.ds(0, CH)], prows_v, sem).wait()
        pltpu.make_async_copy(ubias_hbm.at[pl.ds(0, CH)], ub_v, sem2).wait()
        pltpu.make_async_copy(pbias_hbm.at[pl.ds(0, CH)], pb_v, sem2).wait()

        def body(rl, carry):
            acc = urows_v[rl, pl.ds(0, L)] * prows_v[rl, pl.ds(0, L)]
            for k in range(1, D // L):
                acc = acc + urows_v[rl, pl.ds(L * k, L)] * prows_v[rl, pl.ds(L * k, L)]
            tot = plsc.cumsum(acc)
            plsc.store_scatter(out_v, [jnp.full((L,), cbase + rl, jnp.int32)],
                               tot, mask=last_lane)
            return carry

        lax.fori_loop(0, CH, body, 0)

        def bias_body(g, carry):
            rows = lanes + g * L
            ub = plsc.load_gather(ub_v, [rows, zeros])
            pb = plsc.load_gather(pb_v, [rows, zeros])
            sl = pl.ds(cbase + g * L, L)
            out_v[sl] = out_v[sl] + ub + pb
            return carry

        lax.fori_loop(0, CHG, bias_body, 0)
        return carry

    lax.fori_loop(0, NCH, chunk, 0)
    pltpu.sync_copy(out_v, out_hbm.at[pl.ds(base, BPW)])


@jax.jit
def _run(uidx, pidx, user_emb, place_emb, user_bias, place_bias):
    mesh = plsc.VectorSubcoreMesh(core_axis_name="c", subcore_axis_name="s")
    kern = functools.partial(
        pl.kernel,
        out_type=jax.ShapeDtypeStruct((B,), jnp.float32),
        mesh=mesh,
        compiler_params=pltpu.CompilerParams(needs_layout_passes=False),
        scratch_types=[
            pltpu.VMEM((BPW,), jnp.int32),
            pltpu.VMEM((BPW,), jnp.int32),
            pltpu.VMEM((CH, D), jnp.float32),
            pltpu.VMEM((CH, D), jnp.float32),
            pltpu.VMEM((CH, 1), jnp.float32),
            pltpu.VMEM((CH, 1), jnp.float32),
            pltpu.VMEM((BPW,), jnp.float32),
            pltpu.SemaphoreType.DMA,
            pltpu.SemaphoreType.DMA,
        ],
    )(_sc_body)
    return kern(uidx, pidx, user_emb, place_emb, user_bias, place_bias)


def kernel(inputs, user_emb, place_emb, user_bias, place_bias):
    uidx = inputs[:, 0]
    pidx = inputs[:, 1]
    out = _run(uidx, pidx, user_emb, place_emb, user_bias, place_bias)
    return out.reshape(B, 1)
